# baseline (speedup 1.0000x reference)
import functools
import math

import jax
import jax.numpy as jnp
from jax import lax
from jax.experimental import pallas as pl
from jax.experimental.pallas import tpu as pltpu
from jax.experimental.pallas import tpu_sc as plsc

D = 256
DW = D // 2


_BN = 10000


def _mm_body(x_ref, wlo_ref, whi_ref, y_ref):
    x = x_ref[...]
    for k in range(3):
        lo = jnp.dot(x, wlo_ref[k], preferred_element_type=jnp.float32)
        hi = jnp.dot(x, whi_ref[k], preferred_element_type=jnp.float32)
        lo16 = lax.bitcast_convert_type(
            lo.astype(jnp.bfloat16), jnp.uint16).astype(jnp.int32)
        hi16 = lax.bitcast_convert_type(
            hi.astype(jnp.bfloat16), jnp.uint16).astype(jnp.int32)
        y_ref[k] = lo16 | (hi16 << 16)


def _compute_y_packed(inp, wlo, whi):
    n = inp.shape[0]
    return pl.pallas_call(
        _mm_body,
        grid=(n // _BN,),
        in_specs=[
            pl.BlockSpec((_BN, D), lambda i: (i, 0)),
            pl.BlockSpec((3, D, DW), lambda i: (0, 0, 0)),
            pl.BlockSpec((3, D, DW), lambda i: (0, 0, 0)),
        ],
        out_specs=pl.BlockSpec((3, _BN, DW), lambda i: (0, i, 0)),
        out_shape=jax.ShapeDtypeStruct((3, n, DW), jnp.int32),
    )(inp, wlo, whi)


_R = 80
_LANES = 16


def _gather_sum_call(ypacked, idx0, idx1, idx2, n):
    info = plsc.get_sparse_core_info()
    nw = info.num_cores * info.num_subcores
    chunks = n // _R
    max_c = (chunks + nw - 1) // nw
    rem = chunks - (max_c - 1) * nw
    ipw = max_c * _R

    mesh = plsc.VectorSubcoreMesh(core_axis_name="c", subcore_axis_name="s")

    @functools.partial(
        pl.kernel,
        out_type=jax.ShapeDtypeStruct((n, D), jnp.float32),
        mesh=mesh,
        scratch_types=[
            pltpu.VMEM((3 * ipw,), jnp.int32),
            pltpu.VMEM((2, 3 * _R, DW), jnp.int32),
            pltpu.VMEM((2, _R, D), jnp.float32),
            pltpu.SemaphoreType.DMA,
            pltpu.SemaphoreType.DMA,
            pltpu.SemaphoreType.DMA,
            pltpu.SemaphoreType.DMA,
        ],
    )
    def gather_sum(y_hbm, i0_hbm, i1_hbm, i2_hbm, out_hbm, idx_v, rows_v,
                   out_v, sg0, sg1, so0, so1):
        wid = lax.axis_index("s") * info.num_cores + lax.axis_index("c")
        nc = jnp.where(wid < rem, max_c, max_c - 1)
        c0 = (max_c - 1) * wid + jnp.minimum(wid, rem)

        base = jnp.minimum(c0 * _R, n - ipw)
        shift = c0 * _R - base
        for k, ref in enumerate((i0_hbm, i1_hbm, i2_hbm)):
            pltpu.sync_copy(ref.at[pl.ds(base, ipw)],
                            idx_v.at[pl.ds(k * ipw, ipw)])

        def start_gather(j, buf, sem):
            for k in range(3):
                pltpu.async_copy(
                    y_hbm.at[idx_v.at[pl.ds(k * ipw + shift + j * _R, _R)]],
                    rows_v.at[buf].at[pl.ds(k * _R, _R)], sem)

        def wait_gather(buf, sem):
            for k in range(3):
                pltpu.make_async_copy(
                    y_hbm.at[idx_v.at[pl.ds(0, _R)]],
                    rows_v.at[buf].at[pl.ds(k * _R, _R)], sem).wait()

        def start_out(j, buf, sem):
            pltpu.async_copy(
                out_v.at[buf], out_hbm.at[pl.ds((c0 + j) * _R, _R)], sem)

        def wait_out(buf, sem):
            pltpu.make_async_copy(
                out_v.at[buf], out_hbm.at[pl.ds(0, _R)], sem).wait()

        def compute(buf):
            @plsc.parallel_loop(0, _R, 1, unroll=4)
            def row_body(r):
                for h in range(DW // _LANES):
                    s = pl.ds(h * _LANES, _LANES)
                    w0 = rows_v[buf, r, s]
                    w1 = rows_v[buf, _R + r, s]
                    w2 = rows_v[buf, 2 * _R + r, s]
                    bc = lambda v: lax.bitcast_convert_type(v, jnp.float32)
                    lo = bc(w0 << 16) + bc(w1 << 16) + bc(w2 << 16)
                    mask = jnp.int32(-65536)
                    hi = bc(w0 & mask) + bc(w1 & mask) + bc(w2 & mask)
                    out_v[buf, r, pl.ds(2 * h * _LANES, _LANES)] = lo
                    out_v[buf, r, pl.ds((2 * h + 1) * _LANES, _LANES)] = hi

        start_gather(0, 0, sg0)

        def stage(j, buf, sg_this, sg_next, so_this):
            @pl.when(j < nc)
            def _():
                @pl.when(j + 1 < nc)
                def _():
                    start_gather(j + 1, 1 - buf, sg_next)

                wait_gather(buf, sg_this)

                @pl.when(j >= 2)
                def _():
                    wait_out(buf, so_this)

                compute(buf)
                start_out(j, buf, so_this)

        def pair_body(t, carry):
            stage(2 * t, 0, sg0, sg1, so0)
            stage(2 * t + 1, 1, sg1, sg0, so1)
            return carry

        lax.fori_loop(0, (max_c + 1) // 2, pair_body, 0)

        wait_out(0, so0)
        wait_out(1, so1)

    return gather_sum(ypacked, idx0, idx1, idx2)


def kernel(input, weights, idxs, idxs_bw):
    n = input.shape[0]
    w3 = weights.reshape(3, D, D)

    j = jnp.arange(DW)
    colmap_lo = 32 * (j // 16) + (j % 16)
    wlo = w3[:, :, colmap_lo]
    whi = w3[:, :, colmap_lo + 16]

    yp = _compute_y_packed(input, wlo, whi)
    ypflat = yp.reshape(3 * n, DW)

    idx32 = idxs.astype(jnp.int32)
    idx0 = idx32[:, 0]
    idx1 = idx32[:, 1] + n
    idx2 = idx32[:, 2] + 2 * n

    return _gather_sum_call(ypflat, idx0, idx1, idx2, n)


# speedup vs baseline: 1.0001x; 1.0001x over previous
import functools
import math

import jax
import jax.numpy as jnp
from jax import lax
from jax.experimental import pallas as pl
from jax.experimental.pallas import tpu as pltpu
from jax.experimental.pallas import tpu_sc as plsc

D = 256
DW = D // 2


_BN = 5000


def _mm_body(x_ref, wlo_ref, whi_ref, y_ref):
    x = x_ref[...]
    for k in range(3):
        lo = jnp.dot(x, wlo_ref[k], preferred_element_type=jnp.float32)
        hi = jnp.dot(x, whi_ref[k], preferred_element_type=jnp.float32)
        lo16 = lax.bitcast_convert_type(
            lo.astype(jnp.bfloat16), jnp.uint16).astype(jnp.int32)
        hi16 = lax.bitcast_convert_type(
            hi.astype(jnp.bfloat16), jnp.uint16).astype(jnp.int32)
        y_ref[k] = lo16 | (hi16 << 16)


def _compute_y_packed(inp, wlo, whi):
    n = inp.shape[0]
    return pl.pallas_call(
        _mm_body,
        grid=(n // _BN,),
        in_specs=[
            pl.BlockSpec((_BN, D), lambda i: (i, 0)),
            pl.BlockSpec((3, D, DW), lambda i: (0, 0, 0)),
            pl.BlockSpec((3, D, DW), lambda i: (0, 0, 0)),
        ],
        out_specs=pl.BlockSpec((3, _BN, DW), lambda i: (0, i, 0)),
        out_shape=jax.ShapeDtypeStruct((3, n, DW), jnp.int32),
    )(inp, wlo, whi)


_R = 80
_LANES = 16


def _gather_sum_call(ypacked, idx0, idx1, idx2, n):
    info = plsc.get_sparse_core_info()
    nw = info.num_cores * info.num_subcores
    chunks = n // _R
    max_c = (chunks + nw - 1) // nw
    rem = chunks - (max_c - 1) * nw
    ipw = max_c * _R

    mesh = plsc.VectorSubcoreMesh(core_axis_name="c", subcore_axis_name="s")

    @functools.partial(
        pl.kernel,
        out_type=jax.ShapeDtypeStruct((n, D), jnp.float32),
        mesh=mesh,
        scratch_types=[
            pltpu.VMEM((3 * ipw,), jnp.int32),
            pltpu.VMEM((2, 3 * _R, DW), jnp.int32),
            pltpu.VMEM((2, _R, D), jnp.float32),
            pltpu.SemaphoreType.DMA,
            pltpu.SemaphoreType.DMA,
            pltpu.SemaphoreType.DMA,
            pltpu.SemaphoreType.DMA,
        ],
    )
    def gather_sum(y_hbm, i0_hbm, i1_hbm, i2_hbm, out_hbm, idx_v, rows_v,
                   out_v, sg0, sg1, so0, so1):
        wid = lax.axis_index("s") * info.num_cores + lax.axis_index("c")
        nc = jnp.where(wid < rem, max_c, max_c - 1)
        c0 = (max_c - 1) * wid + jnp.minimum(wid, rem)

        base = jnp.minimum(c0 * _R, n - ipw)
        shift = c0 * _R - base
        for k, ref in enumerate((i0_hbm, i1_hbm, i2_hbm)):
            pltpu.sync_copy(ref.at[pl.ds(base, ipw)],
                            idx_v.at[pl.ds(k * ipw, ipw)])

        def start_gather(j, buf, sem):
            for k in range(3):
                pltpu.async_copy(
                    y_hbm.at[idx_v.at[pl.ds(k * ipw + shift + j * _R, _R)]],
                    rows_v.at[buf].at[pl.ds(k * _R, _R)], sem)

        def wait_gather(buf, sem):
            for k in range(3):
                pltpu.make_async_copy(
                    y_hbm.at[idx_v.at[pl.ds(0, _R)]],
                    rows_v.at[buf].at[pl.ds(k * _R, _R)], sem).wait()

        def start_out(j, buf, sem):
            pltpu.async_copy(
                out_v.at[buf], out_hbm.at[pl.ds((c0 + j) * _R, _R)], sem)

        def wait_out(buf, sem):
            pltpu.make_async_copy(
                out_v.at[buf], out_hbm.at[pl.ds(0, _R)], sem).wait()

        def compute(buf):
            @plsc.parallel_loop(0, _R, 1, unroll=4)
            def row_body(r):
                for h in range(DW // _LANES):
                    s = pl.ds(h * _LANES, _LANES)
                    w0 = rows_v[buf, r, s]
                    w1 = rows_v[buf, _R + r, s]
                    w2 = rows_v[buf, 2 * _R + r, s]
                    bc = lambda v: lax.bitcast_convert_type(v, jnp.float32)
                    lo = bc(w0 << 16) + bc(w1 << 16) + bc(w2 << 16)
                    mask = jnp.int32(-65536)
                    hi = bc(w0 & mask) + bc(w1 & mask) + bc(w2 & mask)
                    out_v[buf, r, pl.ds(2 * h * _LANES, _LANES)] = lo
                    out_v[buf, r, pl.ds((2 * h + 1) * _LANES, _LANES)] = hi

        start_gather(0, 0, sg0)

        def stage(j, buf, sg_this, sg_next, so_this):
            @pl.when(j < nc)
            def _():
                @pl.when(j + 1 < nc)
                def _():
                    start_gather(j + 1, 1 - buf, sg_next)

                wait_gather(buf, sg_this)

                @pl.when(j >= 2)
                def _():
                    wait_out(buf, so_this)

                compute(buf)
                start_out(j, buf, so_this)

        def pair_body(t, carry):
            stage(2 * t, 0, sg0, sg1, so0)
            stage(2 * t + 1, 1, sg1, sg0, so1)
            return carry

        lax.fori_loop(0, (max_c + 1) // 2, pair_body, 0)

        wait_out(0, so0)
        wait_out(1, so1)

    return gather_sum(ypacked, idx0, idx1, idx2)


def kernel(input, weights, idxs, idxs_bw):
    n = input.shape[0]
    w3 = weights.reshape(3, D, D)

    j = jnp.arange(DW)
    colmap_lo = 32 * (j // 16) + (j % 16)
    wlo = w3[:, :, colmap_lo]
    whi = w3[:, :, colmap_lo + 16]

    yp = _compute_y_packed(input, wlo, whi)
    ypflat = yp.reshape(3 * n, DW)

    idx32 = idxs.astype(jnp.int32)
    idx0 = idx32[:, 0]
    idx1 = idx32[:, 1] + n
    idx2 = idx32[:, 2] + 2 * n

    return _gather_sum_call(ypflat, idx0, idx1, idx2, n)
